# baseline (speedup 1.0000x reference)
import functools

import jax
import jax.numpy as jnp
from jax import lax
from jax.experimental import pallas as pl
from jax.experimental.pallas import tpu as pltpu
from jax.experimental.pallas import tpu_sc as plsc

LENGTH = 2048
N_PAIRS = 16

_mesh = plsc.VectorSubcoreMesh(core_axis_name="c", subcore_axis_name="s")


@functools.partial(
    pl.kernel,
    mesh=_mesh,
    out_type=jax.ShapeDtypeStruct((N_PAIRS,), jnp.int32),
    scratch_types=[
        pltpu.VMEM((N_PAIRS,), jnp.int32),
        pltpu.VMEM((N_PAIRS,), jnp.int32),
    ],
)
def _trunc_sc(begin_hbm, end_hbm, out_hbm, b_v, e_v):
    cid = lax.axis_index("c")
    sid = lax.axis_index("s")

    @pl.when(jnp.logical_and(cid == 0, sid == 0))
    def _():
        pltpu.sync_copy(begin_hbm, b_v)
        pltpu.sync_copy(end_hbm, e_v)
        e_v[...] = jnp.minimum(b_v[...] + LENGTH, e_v[...])
        pltpu.sync_copy(e_v, out_hbm)


def kernel(values, indices):
    begin = indices[:, 0]
    new_end = _trunc_sc(begin, indices[:, 1])
    return (values, jnp.stack([begin, new_end], axis=1))


# speedup vs baseline: 1.0004x; 1.0004x over previous
import functools

import jax
import jax.numpy as jnp
from jax import lax
from jax.experimental import pallas as pl
from jax.experimental.pallas import tpu as pltpu
from jax.experimental.pallas import tpu_sc as plsc

LENGTH = 2048
N_PAIRS = 16
FLAT = 2 * N_PAIRS

_mesh = plsc.VectorSubcoreMesh(core_axis_name="c", subcore_axis_name="s")


@functools.partial(
    pl.kernel,
    mesh=_mesh,
    out_type=jax.ShapeDtypeStruct((FLAT,), jnp.int32),
    scratch_types=[pltpu.VMEM((FLAT,), jnp.int32)],
)
def _trunc_sc(idx_hbm, out_hbm, scratch):
    cid = lax.axis_index("c")
    sid = lax.axis_index("s")

    @pl.when(jnp.logical_and(cid == 0, sid == 0))
    def _():
        pltpu.sync_copy(idx_hbm, scratch)
        lane = lax.iota(jnp.int32, 16)
        even = lane - (lane & 1)
        for i in range(FLAT // 16):
            x = scratch[pl.ds(16 * i, 16)]
            b = x.at[even].get(mode="promise_in_bounds")
            scratch[pl.ds(16 * i, 16)] = jnp.minimum(x, b + LENGTH)
        pltpu.sync_copy(scratch, out_hbm)


def kernel(values, indices):
    out = _trunc_sc(indices.reshape(FLAT))
    return (values, out.reshape(N_PAIRS, 2))


# speedup vs baseline: 1.0210x; 1.0206x over previous
import functools

import jax
import jax.numpy as jnp
from jax import lax
from jax.experimental import pallas as pl
from jax.experimental.pallas import tpu as pltpu
from jax.experimental.pallas import tpu_sc as plsc

LENGTH = 2048
N_PAIRS = 16
FLAT = 2 * N_PAIRS

_mesh = plsc.VectorSubcoreMesh(core_axis_name="c", subcore_axis_name="s")


@functools.partial(
    pl.kernel,
    mesh=_mesh,
    out_type=jax.ShapeDtypeStruct((FLAT,), jnp.int32),
    scratch_types=[pltpu.VMEM((FLAT,), jnp.int32)],
)
def _trunc_sc(idx_hbm, out_hbm, scratch):
    cid = lax.axis_index("c")
    sid = lax.axis_index("s")

    @pl.when(jnp.logical_and(cid == 0, sid == 0))
    def _():
        pltpu.sync_copy(idx_hbm, scratch)
        lane = lax.iota(jnp.int32, 16)
        even = lane - (lane & 1)
        for i in range(FLAT // 16):
            x = scratch[pl.ds(16 * i, 16)]
            b = x.at[even].get(mode="promise_in_bounds")
            scratch[pl.ds(16 * i, 16)] = jnp.minimum(x, b + LENGTH)
        pltpu.sync_copy(scratch, out_hbm)


def _copy_body(x_ref, o_ref):
    o_ref[...] = x_ref[...]


def _tc_copy(values):
    rows, cols = values.shape
    block = 1024
    return pl.pallas_call(
        _copy_body,
        grid=(rows // block,),
        in_specs=[pl.BlockSpec((block, cols), lambda i: (i, 0))],
        out_specs=pl.BlockSpec((block, cols), lambda i: (i, 0)),
        out_shape=jax.ShapeDtypeStruct(values.shape, values.dtype),
    )(values)


def kernel(values, indices):
    out = _trunc_sc(indices.reshape(FLAT))
    return (_tc_copy(values), out.reshape(N_PAIRS, 2))
